# baseline (speedup 1.0000x reference)
import jax
import jax.numpy as jnp
from jax import lax
from jax.experimental import pallas as pl
from jax.experimental.pallas import tpu as pltpu
from jax.experimental.pallas import tpu_sc as plsc

B = 16384
D = 64
L = 16
NC = 2
NS = 16
NW = NC * NS
B_PER_W = B // NW
CHUNK = 128
N_CHUNKS = B_PER_W // CHUNK


def _sc_body(user_hbm, item_hbm, utab_hbm, itab_hbm, out_hbm,
             uidx_v, iidx_v, urows_v, irows_v, out_v, sem):
    wid = lax.axis_index("s") * NC + lax.axis_index("c")
    base = wid * B_PER_W

    pltpu.sync_copy(user_hbm.at[wid], uidx_v)
    pltpu.sync_copy(item_hbm.at[wid], iidx_v)

    copies = []
    for j in range(N_CHUNKS):
        copies.append(pltpu.async_copy(
            utab_hbm.at[uidx_v.at[j]], urows_v.at[pl.ds(j * CHUNK, CHUNK)],
            sem))
        copies.append(pltpu.async_copy(
            itab_hbm.at[iidx_v.at[j]], irows_v.at[pl.ds(j * CHUNK, CHUNK)],
            sem))
    for cp in copies:
        cp.wait()

    lane = jnp.arange(L, dtype=jnp.int32)

    def group(g, _):
        rows = g * L + lane
        zero = lane * 0
        acc = (plsc.load_gather(urows_v, [rows, zero])
               * plsc.load_gather(irows_v, [rows, zero]))
        for d in range(1, D):
            col = zero + d
            acc = acc + (plsc.load_gather(urows_v, [rows, col])
                         * plsc.load_gather(irows_v, [rows, col]))
        out_v[pl.ds(g * L, L)] = acc
        return 0

    lax.fori_loop(0, B_PER_W // L, group, 0)

    pltpu.sync_copy(out_v, out_hbm.at[pl.ds(base, B_PER_W)])


@jax.jit
def _run(user, item, user_table, item_table):
    mesh = plsc.VectorSubcoreMesh(core_axis_name="c", subcore_axis_name="s")
    kern = pl.kernel(
        _sc_body,
        out_type=jax.ShapeDtypeStruct((B,), jnp.float32),
        mesh=mesh,
        scratch_types=[
            pltpu.VMEM((N_CHUNKS, CHUNK), jnp.int32),
            pltpu.VMEM((N_CHUNKS, CHUNK), jnp.int32),
            pltpu.VMEM((B_PER_W, D), jnp.float32),
            pltpu.VMEM((B_PER_W, D), jnp.float32),
            pltpu.VMEM((B_PER_W,), jnp.float32),
            pltpu.SemaphoreType.DMA,
        ],
        compiler_params=pltpu.CompilerParams(
            needs_layout_passes=False, use_tc_tiling_on_sc=False),
    )
    u3 = user.astype(jnp.int32).reshape(NW, N_CHUNKS, CHUNK)
    i3 = item.astype(jnp.int32).reshape(NW, N_CHUNKS, CHUNK)
    return kern(u3, i3, user_table, item_table)


def kernel(user, item, user_table, item_table):
    return _run(user, item, user_table, item_table)


# speedup vs baseline: 1.0023x; 1.0023x over previous
import jax
import jax.numpy as jnp
from jax import lax
from jax.experimental import pallas as pl
from jax.experimental.pallas import tpu as pltpu
from jax.experimental.pallas import tpu_sc as plsc

B = 16384
D = 64
L = 16
NC = 2
NS = 16
NW = NC * NS
B_PER_W = B // NW
CHUNK = 128
N_CHUNKS = B_PER_W // CHUNK


def _gather_body(idx_hbm, tab_hbm, emb_hbm, idx_v, rows_v, sem):
    wid = lax.axis_index("s") * NC + lax.axis_index("c")
    base = wid * B_PER_W

    pltpu.sync_copy(idx_hbm.at[wid], idx_v)
    copies = []
    for j in range(N_CHUNKS):
        copies.append(pltpu.async_copy(
            tab_hbm.at[idx_v.at[j]], rows_v.at[pl.ds(j * CHUNK, CHUNK)],
            sem))
    for cp in copies:
        cp.wait()
    pltpu.sync_copy(rows_v, emb_hbm.at[pl.ds(base, B_PER_W)])


def _dot_body(idx_hbm, tab_hbm, emb_hbm, out_hbm,
              idx_v, urows_v, irows_v, out_v, sem):
    wid = lax.axis_index("s") * NC + lax.axis_index("c")
    base = wid * B_PER_W

    pltpu.sync_copy(idx_hbm.at[wid], idx_v)
    copies = [pltpu.async_copy(emb_hbm.at[pl.ds(base, B_PER_W)], irows_v, sem)]
    for j in range(N_CHUNKS):
        copies.append(pltpu.async_copy(
            tab_hbm.at[idx_v.at[j]], urows_v.at[pl.ds(j * CHUNK, CHUNK)],
            sem))
    for cp in copies:
        cp.wait()

    lane = jnp.arange(L, dtype=jnp.int32)

    def group(g, _):
        rows = g * L + lane
        zero = lane * 0
        acc = (plsc.load_gather(urows_v, [rows, zero])
               * plsc.load_gather(irows_v, [rows, zero]))
        for d in range(1, D):
            col = zero + d
            acc = acc + (plsc.load_gather(urows_v, [rows, col])
                         * plsc.load_gather(irows_v, [rows, col]))
        out_v[pl.ds(g * L, L)] = acc
        return 0

    lax.fori_loop(0, B_PER_W // L, group, 0)

    pltpu.sync_copy(out_v, out_hbm.at[pl.ds(base, B_PER_W)])


@jax.jit
def _run(user, item, user_table, item_table):
    mesh = plsc.VectorSubcoreMesh(core_axis_name="c", subcore_axis_name="s")
    params = pltpu.CompilerParams(
        needs_layout_passes=False, use_tc_tiling_on_sc=False)
    gather_k = pl.kernel(
        _gather_body,
        out_type=jax.ShapeDtypeStruct((B, D), jnp.float32),
        mesh=mesh,
        scratch_types=[
            pltpu.VMEM((N_CHUNKS, CHUNK), jnp.int32),
            pltpu.VMEM((B_PER_W, D), jnp.float32),
            pltpu.SemaphoreType.DMA,
        ],
        compiler_params=params,
    )
    dot_k = pl.kernel(
        _dot_body,
        out_type=jax.ShapeDtypeStruct((B,), jnp.float32),
        mesh=mesh,
        scratch_types=[
            pltpu.VMEM((N_CHUNKS, CHUNK), jnp.int32),
            pltpu.VMEM((B_PER_W, D), jnp.float32),
            pltpu.VMEM((B_PER_W, D), jnp.float32),
            pltpu.VMEM((B_PER_W,), jnp.float32),
            pltpu.SemaphoreType.DMA,
        ],
        compiler_params=params,
    )
    u3 = user.astype(jnp.int32).reshape(NW, N_CHUNKS, CHUNK)
    i3 = item.astype(jnp.int32).reshape(NW, N_CHUNKS, CHUNK)
    iemb = gather_k(i3, item_table, )
    return dot_k(u3, user_table, iemb)


def kernel(user, item, user_table, item_table):
    return _run(user, item, user_table, item_table)


# speedup vs baseline: 2.2632x; 2.2581x over previous
import jax
import jax.numpy as jnp
from jax import lax
from jax.experimental import pallas as pl
from jax.experimental.pallas import tpu as pltpu
from jax.experimental.pallas import tpu_sc as plsc

B = 16384
D = 64
TPB = 8
L = 16
NC = 2
NS = 16
NW = NC * NS
B_PER_W = B // NW
CHUNK = 16
N_CHUNKS = B_PER_W // CHUNK
NBUF = 2


def _sc_body(utid_hbm, itid_hbm, uoff_hbm, ioff_hbm,
             utab_hbm, itab_hbm, out_hbm,
             utid_s, itid_s, uoff_v, ioff_v, ub_v, ib_v, out_v, usem, isem):
    wid = lax.axis_index("s") * NC + lax.axis_index("c")

    pltpu.sync_copy(utid_hbm.at[wid], utid_s)
    pltpu.sync_copy(itid_hbm.at[wid], itid_s)
    pltpu.sync_copy(uoff_hbm.at[wid], uoff_v)
    pltpu.sync_copy(ioff_hbm.at[wid], ioff_v)

    def fire(j, buf):
        def one(g, _):
            ut16 = utid_s[pl.ds(j * CHUNK + g * L, L)]
            it16 = itid_s[pl.ds(j * CHUNK + g * L, L)]
            for k in range(L):
                pltpu.async_copy(
                    utab_hbm.at[ut16[k]], ub_v.at[buf, g * L + k], usem)
                pltpu.async_copy(
                    itab_hbm.at[it16[k]], ib_v.at[buf, g * L + k], isem)
            return 0
        lax.fori_loop(0, CHUNK // L, one, 0)

    def wait(j, buf):
        def one(g, _):
            ut16 = utid_s[pl.ds(j * CHUNK + g * L, L)]
            it16 = itid_s[pl.ds(j * CHUNK + g * L, L)]
            for k in range(L):
                pltpu.make_async_copy(
                    utab_hbm.at[ut16[k]], ub_v.at[buf, g * L + k],
                    usem).wait()
                pltpu.make_async_copy(
                    itab_hbm.at[it16[k]], ib_v.at[buf, g * L + k],
                    isem).wait()
            return 0
        lax.fori_loop(0, CHUNK // L, one, 0)

    fire(0, 0)

    lane = jnp.arange(L, dtype=jnp.int32)

    def chunk_body(j, _):
        buf = j % NBUF

        @pl.when(j + 1 < N_CHUNKS)
        def _():
            fire(j + 1, (j + 1) % NBUF)

        wait(j, buf)

        bufv = lane * 0 + buf
        for g in range(CHUNK // L):
            slot = lane + g * L
            base = j * CHUNK + g * L
            uoff = uoff_v[pl.ds(base, L)]
            ioff = ioff_v[pl.ds(base, L)]
            zero = lane * 0
            acc = (plsc.load_gather(ub_v, [bufv, slot, uoff, zero])
                   * plsc.load_gather(ib_v, [bufv, slot, ioff, zero]))
            for d in range(1, D):
                col = zero + d
                acc = acc + (plsc.load_gather(ub_v, [bufv, slot, uoff, col])
                             * plsc.load_gather(ib_v, [bufv, slot, ioff, col]))
            out_v[pl.ds(base, L)] = acc
        return 0

    lax.fori_loop(0, N_CHUNKS, chunk_body, 0)

    pltpu.sync_copy(out_v, out_hbm.at[pl.ds(wid * B_PER_W, B_PER_W)])


@jax.jit
def _run(user, item, user_table, item_table):
    mesh = plsc.VectorSubcoreMesh(core_axis_name="c", subcore_axis_name="s")
    kern = pl.kernel(
        _sc_body,
        out_type=jax.ShapeDtypeStruct((B,), jnp.float32),
        mesh=mesh,
        scratch_types=[
            pltpu.VMEM((B_PER_W,), jnp.int32),
            pltpu.VMEM((B_PER_W,), jnp.int32),
            pltpu.VMEM((B_PER_W,), jnp.int32),
            pltpu.VMEM((B_PER_W,), jnp.int32),
            pltpu.VMEM((NBUF, CHUNK, TPB, D), jnp.float32),
            pltpu.VMEM((NBUF, CHUNK, TPB, D), jnp.float32),
            pltpu.VMEM((B_PER_W,), jnp.float32),
            pltpu.SemaphoreType.DMA,
            pltpu.SemaphoreType.DMA,
        ],
        compiler_params=pltpu.CompilerParams(needs_layout_passes=False),
    )
    u = user.astype(jnp.int32)
    i = item.astype(jnp.int32)
    return kern(
        (u >> 3).reshape(NW, B_PER_W),
        (i >> 3).reshape(NW, B_PER_W),
        (u & 7).reshape(NW, B_PER_W),
        (i & 7).reshape(NW, B_PER_W),
        user_table.reshape(1000000 // TPB, TPB, D),
        item_table.reshape(1000000 // TPB, TPB, D),
    )


def kernel(user, item, user_table, item_table):
    return _run(user, item, user_table, item_table)
